# baseline (speedup 1.0000x reference)
import jax
import jax.numpy as jnp
from jax import lax
from jax.experimental import pallas as pl
from jax.experimental.pallas import tpu as pltpu
from jax.experimental.pallas import tpu_sc as plsc

F = 26
V = 100000
H = 64
B = 16384
NC = 2
NS = 16
L = 16
NW = NC * NS
BPW = B // NW
CB = 128
NCHUNK = BPW // CB
NG = 13

_mesh = plsc.VectorSubcoreMesh(core_axis_name="c", subcore_axis_name="s")


def _cat_sum_body(xt_hbm, *refs):
    tabs = refs[:NG]
    out_hbm = refs[NG]
    xv, idxA, idxB, gbufA, gbufB, acc, semA, semB = refs[NG + 1:]
    wid = lax.axis_index("s") * NC + lax.axis_index("c")
    base = wid * BPW
    pltpu.sync_copy(xt_hbm.at[:, pl.ds(base, BPW)], xv)

    def chunk_body(c, carry):
        cb = c * CB

        def make_idx(f, floc, idxv):
            for s in range(CB // L):
                xc = xv[f, pl.ds(cb + s * L, L)]
                xc = jnp.minimum(jnp.maximum(xc, 0), V - 1)
                idxv[0, pl.ds(s * L, L)] = xc + floc * V

        def process(gbuf, first):
            def rbody(r, carry2):
                for cc in range(H // L):
                    g = gbuf[r, pl.ds(cc * L, L)]
                    if first:
                        acc[r, pl.ds(cc * L, L)] = g
                    else:
                        plsc.addupdate(acc.at[r, pl.ds(cc * L, L)], g)
                return carry2

            lax.fori_loop(0, CB, rbody, 0, unroll=2)

        for g in range(NG):
            make_idx(2 * g, 0, idxA)
            cpA = pltpu.async_copy(tabs[g].at[idxA.at[0]], gbufA, semA)
            make_idx(2 * g + 1, 1, idxB)
            cpB = pltpu.async_copy(tabs[g].at[idxB.at[0]], gbufB, semB)
            cpA.wait()
            process(gbufA, first=(g == 0))
            cpB.wait()
            process(gbufB, first=False)

        pltpu.sync_copy(acc, out_hbm.at[pl.ds(base + cb, CB), :])
        return carry

    lax.fori_loop(0, NCHUNK, chunk_body, 0)


_cat_sum = pl.kernel(
    _cat_sum_body,
    out_type=jax.ShapeDtypeStruct((B, H), jnp.float32),
    mesh=_mesh,
    compiler_params=pltpu.CompilerParams(use_tc_tiling_on_sc=False),
    scratch_types=[
        pltpu.VMEM((F, BPW), jnp.int32),
        pltpu.VMEM((1, CB), jnp.int32),
        pltpu.VMEM((1, CB), jnp.int32),
        pltpu.VMEM((CB, H), jnp.float32),
        pltpu.VMEM((CB, H), jnp.float32),
        pltpu.VMEM((CB, H), jnp.float32),
        pltpu.SemaphoreType.DMA,
        pltpu.SemaphoreType.DMA,
    ],
)


def kernel(x, tables):
    xt = jnp.transpose(x.astype(jnp.int32))
    tabs = [tables[2 * g:2 * g + 2].reshape(2 * V, H) for g in range(NG)]
    return _cat_sum(xt, *tabs)


# speedup vs baseline: 1.5768x; 1.5768x over previous
import jax
import jax.numpy as jnp
from jax import lax
from jax.experimental import pallas as pl
from jax.experimental.pallas import tpu as pltpu
from jax.experimental.pallas import tpu_sc as plsc

F = 26
V = 100000
H = 64
B = 16384
NC = 2
NS = 16
L = 16
NW = NC * NS
BPW = B // NW
CB = 128
NCHUNK = BPW // CB

_mesh = plsc.VectorSubcoreMesh(core_axis_name="c", subcore_axis_name="s")


def _cat_sum_body(xt_hbm, tab_hbm, out_hbm,
                  xv, idxA, idxB, gbufA, gbufB, acc, semA, semB):
    wid = lax.axis_index("s") * NC + lax.axis_index("c")
    base = wid * BPW
    pltpu.sync_copy(xt_hbm.at[:, pl.ds(base, BPW)], xv)

    def chunk_body(c, carry):
        cb = c * CB

        def make_idx(f, idxv):
            for s in range(CB // L):
                xc = xv[f, pl.ds(cb + s * L, L)]
                idxv[0, pl.ds(s * L, L)] = jnp.minimum(
                    jnp.maximum(xc, 0), V - 1)

        def fire(f, idxv, gbuf, sem):
            return pltpu.async_copy(
                tab_hbm.at[f].at[idxv.at[0]], gbuf, sem)

        def wait(idxv, gbuf, sem):
            pltpu.make_async_copy(
                tab_hbm.at[0].at[idxv.at[0]], gbuf, sem).wait()

        def process(gbuf, first):
            def rbody(r, carry2):
                for cc in range(H // L):
                    g = gbuf[r, pl.ds(cc * L, L)]
                    if first:
                        acc[r, pl.ds(cc * L, L)] = g
                    else:
                        plsc.addupdate(acc.at[r, pl.ds(cc * L, L)], g)
                return carry2

            lax.fori_loop(0, CB, rbody, 0, unroll=2)

        make_idx(0, idxA)
        pltpu.async_copy(tab_hbm.at[0].at[idxA.at[0]], acc, semA).wait()
        make_idx(1, idxA)
        fire(1, idxA, gbufA, semA)
        make_idx(2, idxB)
        fire(2, idxB, gbufB, semB)
        wait(idxA, gbufA, semA)
        process(gbufA, first=False)

        def pair_body(k, carry2):
            fa = 2 * k + 3
            make_idx(fa, idxA)
            fire(fa, idxA, gbufA, semA)
            wait(idxB, gbufB, semB)
            process(gbufB, first=False)
            make_idx(fa + 1, idxB)
            fire(fa + 1, idxB, gbufB, semB)
            wait(idxA, gbufA, semA)
            process(gbufA, first=False)
            return carry2

        lax.fori_loop(0, (F - 4) // 2, pair_body, 0)
        make_idx(F - 1, idxA)
        fire(F - 1, idxA, gbufA, semA)
        wait(idxB, gbufB, semB)
        process(gbufB, first=False)
        wait(idxA, gbufA, semA)
        process(gbufA, first=False)

        pltpu.sync_copy(acc, out_hbm.at[pl.ds(base + cb, CB), :])
        return carry

    lax.fori_loop(0, NCHUNK, chunk_body, 0)


_cat_sum = pl.kernel(
    _cat_sum_body,
    out_type=jax.ShapeDtypeStruct((B, H), jnp.float32),
    mesh=_mesh,
    compiler_params=pltpu.CompilerParams(use_tc_tiling_on_sc=False),
    scratch_types=[
        pltpu.VMEM((F, BPW), jnp.int32),
        pltpu.VMEM((1, CB), jnp.int32),
        pltpu.VMEM((1, CB), jnp.int32),
        pltpu.VMEM((CB, H), jnp.float32),
        pltpu.VMEM((CB, H), jnp.float32),
        pltpu.VMEM((CB, H), jnp.float32),
        pltpu.SemaphoreType.DMA,
        pltpu.SemaphoreType.DMA,
    ],
)


def kernel(x, tables):
    xt = jnp.transpose(x.astype(jnp.int32))
    return _cat_sum(xt, tables)
